# Initial kernel scaffold; baseline (speedup 1.0000x reference)
import jax
import jax.numpy as jnp
from jax.experimental import pallas as pl


def kernel(x, pe):
    raise NotImplementedError("write your pallas kernel here")


# speedup vs baseline: 2.7626x; 2.7626x over previous
import jax
import jax.numpy as jnp
from jax import lax
from jax.experimental import pallas as pl
from jax.experimental.pallas import tpu as pltpu
from jax.experimental.pallas import tpu_sc as plsc

NC = 2
NS = 16
NW = NC * NS
CHUNK = 128


def _gather_kernel(n_rows, d):
    rows_per_w = n_rows // NW
    steps = rows_per_w // CHUNK
    idx_rows = rows_per_w // CHUNK

    mesh = plsc.VectorSubcoreMesh(
        core_axis_name="c", subcore_axis_name="s",
        num_cores=NC, num_subcores=NS)

    def body(x_hbm, pe_hbm, out_hbm, idx_v, rows_v, sem):
        c = lax.axis_index("c")
        s = lax.axis_index("s")
        wid = s * NC + c
        pltpu.sync_copy(x_hbm.at[pl.ds(wid * idx_rows, idx_rows)], idx_v)

        def step(j, _):
            cp = pltpu.async_copy(pe_hbm.at[idx_v.at[j]], rows_v, sem)
            cp.wait()
            pltpu.sync_copy(
                rows_v, out_hbm.at[pl.ds(wid * rows_per_w + j * CHUNK, CHUNK)])
            return _

        lax.fori_loop(0, steps, step, None)

    return pl.kernel(
        body,
        out_type=jax.ShapeDtypeStruct((n_rows, d), jnp.float32),
        mesh=mesh,
        scratch_types=[
            pltpu.VMEM((idx_rows, CHUNK), jnp.int32),
            pltpu.VMEM((CHUNK, d), jnp.float32),
            pltpu.SemaphoreType.DMA,
        ],
        compiler_params=pltpu.CompilerParams(use_tc_tiling_on_sc=False),
    )


def kernel(x, pe):
    b, s = x.shape
    d = pe.shape[1]
    n_rows = b * s
    x2d = x.reshape(n_rows // CHUNK, CHUNK)
    out = _gather_kernel(n_rows, d)(x2d, pe)
    return out.reshape(b, s, d)


# speedup vs baseline: 2.7728x; 1.0037x over previous
import jax
import jax.numpy as jnp
from jax import lax
from jax.experimental import pallas as pl
from jax.experimental.pallas import tpu as pltpu
from jax.experimental.pallas import tpu_sc as plsc

NC = 2
NS = 16
NW = NC * NS
CHUNK = 128
GROUP = 512
K = GROUP // CHUNK


def _gather_kernel(n_rows, d):
    rows_per_w = n_rows // NW
    idx_rows = rows_per_w // CHUNK
    n_groups = rows_per_w // GROUP

    mesh = plsc.VectorSubcoreMesh(
        core_axis_name="c", subcore_axis_name="s",
        num_cores=NC, num_subcores=NS)

    def body(x_hbm, pe_hbm, out_hbm, idx_v, buf0, buf1, gs0, gs1, os0, os1):
        c = lax.axis_index("c")
        s = lax.axis_index("s")
        wid = s * NC + c
        base_row = wid * rows_per_w
        pltpu.sync_copy(x_hbm.at[pl.ds(wid * idx_rows, idx_rows)], idx_v)

        bufs = (buf0, buf1)
        gsems = (gs0, gs1)
        osems = (os0, os1)

        def fire(g, p):
            return [
                pltpu.async_copy(
                    pe_hbm.at[idx_v.at[g * K + j]],
                    bufs[p].at[pl.ds(j * CHUNK, CHUNK)],
                    gsems[p])
                for j in range(K)
            ]

        def out_copy(g, p):
            return pltpu.async_copy(
                bufs[p], out_hbm.at[pl.ds(base_row + g * GROUP, GROUP)],
                osems[p])

        def step(go, _):
            g0 = go * 2
            g1 = go * 2 + 1
            cps0 = fire(g0, 0)
            cps1 = fire(g1, 1)
            for cp in cps0:
                cp.wait()
            oc0 = out_copy(g0, 0)
            for cp in cps1:
                cp.wait()
            oc1 = out_copy(g1, 1)
            oc0.wait()
            oc1.wait()
            return _

        lax.fori_loop(0, n_groups // 2, step, None)

    return pl.kernel(
        body,
        out_type=jax.ShapeDtypeStruct((n_rows, d), jnp.float32),
        mesh=mesh,
        scratch_types=[
            pltpu.VMEM((idx_rows, CHUNK), jnp.int32),
            pltpu.VMEM((GROUP, d), jnp.float32),
            pltpu.VMEM((GROUP, d), jnp.float32),
            pltpu.SemaphoreType.DMA,
            pltpu.SemaphoreType.DMA,
            pltpu.SemaphoreType.DMA,
            pltpu.SemaphoreType.DMA,
        ],
        compiler_params=pltpu.CompilerParams(use_tc_tiling_on_sc=False),
    )


def kernel(x, pe):
    b, s = x.shape
    d = pe.shape[1]
    n_rows = b * s
    x2d = x.reshape(n_rows // CHUNK, CHUNK)
    out = _gather_kernel(n_rows, d)(x2d, pe)
    return out.reshape(b, s, d)


# speedup vs baseline: 4.6922x; 1.6922x over previous
import jax
import jax.numpy as jnp
from jax import lax
from jax.experimental import pallas as pl
from jax.experimental.pallas import tpu as pltpu
from jax.experimental.pallas import tpu_sc as plsc

NC = 2
NS = 16
NW = NC * NS
CHUNK = 128
GROUP = 512
K = GROUP // CHUNK


def _gather_kernel(n_rows, v, d):
    rows_per_w = n_rows // NW
    idx_rows = rows_per_w // CHUNK
    n_groups = rows_per_w // GROUP

    mesh = plsc.VectorSubcoreMesh(
        core_axis_name="c", subcore_axis_name="s",
        num_cores=NC, num_subcores=NS)

    def body(x_hbm, pe_hbm, out_hbm, idx_v, buf0, buf1, pe_sh,
             gs0, gs1, os0, os1):
        c = lax.axis_index("c")
        s = lax.axis_index("s")
        wid = s * NC + c
        base_row = wid * rows_per_w

        @pl.when(s == 0)
        def _():
            pltpu.sync_copy(pe_hbm, pe_sh)

        pltpu.sync_copy(x_hbm.at[pl.ds(wid * idx_rows, idx_rows)], idx_v)
        plsc.subcore_barrier()

        bufs = (buf0, buf1)
        gsems = (gs0, gs1)
        osems = (os0, os1)

        def fire(g, p):
            return [
                pltpu.async_copy(
                    pe_sh.at[idx_v.at[g * K + j]],
                    bufs[p].at[pl.ds(j * CHUNK, CHUNK)],
                    gsems[p])
                for j in range(K)
            ]

        def out_copy(g, p):
            return pltpu.async_copy(
                bufs[p], out_hbm.at[pl.ds(base_row + g * GROUP, GROUP)],
                osems[p])

        def step(go, _):
            g0 = go * 2
            g1 = go * 2 + 1
            cps0 = fire(g0, 0)
            cps1 = fire(g1, 1)
            for cp in cps0:
                cp.wait()
            oc0 = out_copy(g0, 0)
            for cp in cps1:
                cp.wait()
            oc1 = out_copy(g1, 1)
            oc0.wait()
            oc1.wait()
            return _

        lax.fori_loop(0, n_groups // 2, step, None)

    return pl.kernel(
        body,
        out_type=jax.ShapeDtypeStruct((n_rows, d), jnp.float32),
        mesh=mesh,
        scratch_types=[
            pltpu.VMEM((idx_rows, CHUNK), jnp.int32),
            pltpu.VMEM((GROUP, d), jnp.float32),
            pltpu.VMEM((GROUP, d), jnp.float32),
            pltpu.VMEM_SHARED((v, d), jnp.float32),
            pltpu.SemaphoreType.DMA,
            pltpu.SemaphoreType.DMA,
            pltpu.SemaphoreType.DMA,
            pltpu.SemaphoreType.DMA,
        ],
        compiler_params=pltpu.CompilerParams(use_tc_tiling_on_sc=False),
    )


def kernel(x, pe):
    b, s = x.shape
    d = pe.shape[1]
    n_rows = b * s
    x2d = x.reshape(n_rows // CHUNK, CHUNK)
    out = _gather_kernel(n_rows, pe.shape[0], d)(x2d, pe)
    return out.reshape(b, s, d)


# speedup vs baseline: 4.7013x; 1.0019x over previous
import jax
import jax.numpy as jnp
from jax import lax
from jax.experimental import pallas as pl
from jax.experimental.pallas import tpu as pltpu
from jax.experimental.pallas import tpu_sc as plsc

NC = 2
NS = 16
NW = NC * NS
CHUNK = 128
GROUP = 512
K = GROUP // CHUNK


def _gather_kernel(n_rows, v, d):
    rows_per_w = n_rows // NW
    idx_rows = rows_per_w // CHUNK
    n_groups = rows_per_w // GROUP

    mesh = plsc.VectorSubcoreMesh(
        core_axis_name="c", subcore_axis_name="s",
        num_cores=NC, num_subcores=NS)

    def body(x_hbm, pe_hbm, out_hbm, idx_v, pe_sh, buf0, buf1,
             gs0, gs1, os0, os1):
        c = lax.axis_index("c")
        s = lax.axis_index("s")
        wid = s * NC + c
        base_row = wid * rows_per_w

        @pl.when(s == 0)
        def _():
            pltpu.sync_copy(pe_hbm, pe_sh)

        pltpu.sync_copy(x_hbm.at[pl.ds(wid * idx_rows, idx_rows)], idx_v)
        plsc.subcore_barrier()

        bufs = (buf0, buf1)
        gsems = (gs0, gs1)
        osems = (os0, os1)

        def fire(g, p):
            return [
                pltpu.async_copy(
                    pe_sh.at[idx_v.at[g * K + j]],
                    bufs[p].at[pl.ds(j * CHUNK, CHUNK)],
                    gsems[p])
                for j in range(K)
            ]

        def out_copy(g, p):
            return pltpu.async_copy(
                bufs[p], out_hbm.at[pl.ds(base_row + g * GROUP, GROUP)],
                osems[p])

        def step(go, _):
            g0 = go * 2
            g1 = go * 2 + 1
            cps0 = fire(g0, 0)
            cps1 = fire(g1, 1)
            for cp in cps0:
                cp.wait()
            oc0 = out_copy(g0, 0)
            for cp in cps1:
                cp.wait()
            oc1 = out_copy(g1, 1)
            oc0.wait()
            oc1.wait()
            return _

        lax.fori_loop(0, n_groups // 2, step, None)

    return pl.kernel(
        body,
        out_type=jax.ShapeDtypeStruct((n_rows, d), jnp.float32),
        mesh=mesh,
        scratch_types=[
            pltpu.VMEM((idx_rows, CHUNK), jnp.int32),
            pltpu.VMEM_SHARED((v, d), jnp.float32),
            pltpu.VMEM((GROUP, d), jnp.float32),
            pltpu.VMEM((GROUP, d), jnp.float32),
            pltpu.SemaphoreType.DMA,
            pltpu.SemaphoreType.DMA,
            pltpu.SemaphoreType.DMA,
            pltpu.SemaphoreType.DMA,
        ],
        compiler_params=pltpu.CompilerParams(use_tc_tiling_on_sc=False),
    )


def kernel(x, pe):
    b, s = x.shape
    d = pe.shape[1]
    n_rows = b * s
    x2d = x.reshape(n_rows // CHUNK, CHUNK)
    out = _gather_kernel(n_rows, pe.shape[0], d)(x2d, pe)
    return out.reshape(b, s, d)
